# baseline (speedup 1.0000x reference)
import functools
import math

import jax
import jax.numpy as jnp
from jax import lax
from jax.experimental import pallas as pl
from jax.experimental.pallas import tpu as pltpu
from jax.experimental.pallas import tpu_sc as plsc

N, C, K, H, FF = 16384, 256, 27, 8, 512
DH = C // H
CP = C // 2
NW = 32
CH = 128
NBUF = 6
NCHUNK = (N * K) // (NW * CH)
BN = 128
KB = K * BN


def _sc_gather(table, idx2):
    mesh = plsc.VectorSubcoreMesh(core_axis_name="c", subcore_axis_name="s")

    @functools.partial(
        pl.kernel,
        out_type=jax.ShapeDtypeStruct((N * K, CP), jnp.int32),
        mesh=mesh,
        scratch_types=[
            pltpu.VMEM((NCHUNK, CH), jnp.int32),
        ] + [pltpu.VMEM((CH, CP), jnp.int32) for _ in range(NBUF)]
          + [pltpu.SemaphoreType.DMA for _ in range(NBUF)],
    )
    def gather_kernel(table_hbm, idx_hbm, out_hbm, idx_v, *bufsem):
        bufs = bufsem[:NBUF]
        sems = bufsem[NBUF:]
        wid = lax.axis_index("s") * 2 + lax.axis_index("c")
        base = wid * NCHUNK
        pltpu.sync_copy(idx_hbm.at[wid], idx_v)
        for b in range(NBUF):
            pltpu.make_async_copy(table_hbm.at[idx_v.at[b]], bufs[b], sems[b]).start()

        def do(jj, buf, sem):
            pltpu.make_async_copy(table_hbm.at[idx_v.at[jj]], buf, sem).wait()
            pltpu.sync_copy(buf, out_hbm.at[pl.ds((base + jj) * CH, CH)])

            @pl.when(jj + NBUF < NCHUNK)
            def _():
                pltpu.make_async_copy(
                    table_hbm.at[idx_v.at[jj + NBUF]], buf, sem
                ).start()

        def step(t, carry):
            for b in range(NBUF):
                do(t * NBUF + b, bufs[b], sems[b])
            return carry

        lax.fori_loop(0, NCHUNK // NBUF, step, 0)
        for r in range(NCHUNK - (NCHUNK % NBUF), NCHUNK):
            do(r, bufs[r % NBUF], sems[r % NBUF])

    return gather_kernel(table, idx2)


def _tc_body(vf_ref, co_ref, g_ref, mk_ref, wq_ref, wk_ref, wv_ref, wo_ref,
             w1_ref, w2_ref, pw_ref, pv_ref, pb1_ref, seg_ref, out_ref):
    f32 = jnp.float32
    bf16 = jnp.bfloat16
    dims_tt = (((1,), (1,)), ((), ()))
    dims_nn = (((1,), (0,)), ((), ()))
    pv = pv_ref[...]
    seg = seg_ref[...]

    vf = vf_ref[...]
    q = lax.dot_general(vf.astype(bf16), wq_ref[...], dims_tt,
                        preferred_element_type=f32)
    q = (q + pv[0:1, :]) * (1.0 / math.sqrt(DH))

    coo = co_ref[...].reshape(KB, 3)
    pos = lax.dot_general(coo, pw_ref[...], dims_tt, preferred_element_type=f32)
    pos = jnp.maximum(pos + pv[8:9, :], 0.0).astype(bf16)

    g2 = g_ref[...].reshape(KB, CP)
    fb = pltpu.bitcast(g2, bf16).reshape(KB, C) + pos
    kp = lax.dot_general(fb, wk_ref[...], dims_tt, preferred_element_type=f32)
    vp = lax.dot_general(fb, wv_ref[...], dims_tt, preferred_element_type=f32)

    qk = jnp.broadcast_to(q[None, :, :], (K, BN, C)).reshape(KB, C)
    s = lax.dot_general(qk * kp, seg, dims_nn, preferred_element_type=f32)
    s3 = s.reshape(K, BN, H)
    s3 = jnp.where(mk_ref[...][:, :, None] > 0.5, -1e9, s3)
    m = jnp.max(s3, axis=0, keepdims=True)
    e = jnp.exp(s3 - m)
    attn = (e / jnp.sum(e, axis=0, keepdims=True)).reshape(KB, H)

    ab = lax.dot_general(attn, seg, (((1,), (1,)), ((), ())),
                         preferred_element_type=f32)
    mixed = jnp.sum((ab * vp).reshape(K, BN, C), axis=0)

    att = lax.dot_general((mixed + pv[1:2, :]).astype(bf16), wo_ref[...],
                          dims_tt, preferred_element_type=f32) + pv[2:3, :]

    def ln(x, g_row, b_row):
        mu = jnp.mean(x, axis=-1, keepdims=True)
        d = x - mu
        var = jnp.mean(d * d, axis=-1, keepdims=True)
        return d * lax.rsqrt(var + 1e-5) * g_row + b_row

    x = ln(vf + att, pv[4:5, :], pv[5:6, :])
    ff = jnp.maximum(
        lax.dot_general(x.astype(bf16), w1_ref[...], dims_tt,
                        preferred_element_type=f32)
        + pb1_ref[0:1, :], 0.0)
    f2 = lax.dot_general(ff.astype(bf16), w2_ref[...], dims_tt,
                         preferred_element_type=f32) + pv[3:4, :]
    out_ref[...] = ln(x + f2, pv[6:7, :], pv[7:8, :])


def kernel(voxel_features, key_coords, Wq, Wk, Wv, bq, bk, bv, Wo, bo,
           W1, b1, W2, b2, ln1_g, ln1_b, ln2_g, ln2_b, pos_w, pos_b,
           key_indices, key_mask):
    del bk

    vfb = voxel_features.astype(jnp.bfloat16)
    tb = lax.bitcast_convert_type(
        jnp.stack([vfb[:, :CP], vfb[:, CP:]], axis=-1), jnp.int32)

    idx_t = key_indices.T.astype(jnp.int32).reshape(NW, NCHUNK, CH)
    gathered = _sc_gather(tb, idx_t).reshape(K, N, CP)

    coords_t = key_coords.transpose(2, 0, 1)
    maskf = key_mask.T.astype(jnp.float32)
    zc = jnp.zeros((1, C), jnp.float32)
    pv = jnp.concatenate(
        [bq[None, :], bv[None, :], bo[None, :], b2[None, :],
         ln1_g[None, :], ln1_b[None, :], ln2_g[None, :], ln2_b[None, :],
         pos_b[None, :], zc, zc, zc, zc, zc, zc, zc], axis=0)
    pb1 = jnp.broadcast_to(b1[None, :], (8, FF))
    seg = (jnp.arange(C)[:, None] // DH ==
           jnp.arange(H)[None, :]).astype(jnp.float32)
    wkb = Wk.astype(jnp.bfloat16)
    wvb = Wv.astype(jnp.bfloat16)
    wqb = Wq.astype(jnp.bfloat16)
    wob = Wo.astype(jnp.bfloat16)
    w1b = W1.astype(jnp.bfloat16)
    w2b = W2.astype(jnp.bfloat16)

    grid = N // BN
    full = lambda shape: pl.BlockSpec(shape, lambda i: tuple(0 for _ in shape))
    out = pl.pallas_call(
        _tc_body,
        grid=(grid,),
        in_specs=[
            pl.BlockSpec((BN, C), lambda i: (i, 0)),
            pl.BlockSpec((K, BN, 3), lambda i: (0, i, 0)),
            pl.BlockSpec((K, BN, CP), lambda i: (0, i, 0)),
            pl.BlockSpec((K, BN), lambda i: (0, i)),
            full((C, C)), full((C, C)), full((C, C)), full((C, C)),
            full((FF, C)), full((C, FF)),
            full((C, 3)), full((16, C)), full((8, FF)), full((C, H)),
        ],
        out_specs=pl.BlockSpec((BN, C), lambda i: (i, 0)),
        out_shape=jax.ShapeDtypeStruct((N, C), jnp.float32),
        compiler_params=pltpu.CompilerParams(
            dimension_semantics=("arbitrary",)),
    )(voxel_features, coords_t, gathered, maskf, wqb, wkb, wvb, wob,
      w1b, w2b, pos_w, pv, pb1, seg)
    return out


# speedup vs baseline: 1.0020x; 1.0020x over previous
import functools
import math

import jax
import jax.numpy as jnp
from jax import lax
from jax.experimental import pallas as pl
from jax.experimental.pallas import tpu as pltpu
from jax.experimental.pallas import tpu_sc as plsc

N, C, K, H, FF = 16384, 256, 27, 8, 512
DH = C // H
CP = C // 2
NW = 32
CH = 128
NBUF = 6
NCHUNK = (N * K) // (NW * CH)
BN = 128
KB = K * BN


def _sc_gather(table, idx2):
    mesh = plsc.VectorSubcoreMesh(core_axis_name="c", subcore_axis_name="s")

    @functools.partial(
        pl.kernel,
        out_type=jax.ShapeDtypeStruct((N * K, CP), jnp.int32),
        mesh=mesh,
        scratch_types=[
            pltpu.VMEM((NCHUNK, CH), jnp.int32),
        ] + [pltpu.VMEM((CH, CP), jnp.int32) for _ in range(NBUF)]
          + [pltpu.SemaphoreType.DMA for _ in range(2 * NBUF)],
    )
    def gather_kernel(table_hbm, idx_hbm, out_hbm, idx_v, *bufsem):
        bufs = bufsem[:NBUF]
        gsems = bufsem[NBUF:2 * NBUF]
        ssems = bufsem[2 * NBUF:]
        wid = lax.axis_index("s") * 2 + lax.axis_index("c")
        base = wid * NCHUNK
        pltpu.sync_copy(idx_hbm.at[wid], idx_v)

        LOOK = NBUF - 2

        def gstart(jj, b):
            pltpu.make_async_copy(table_hbm.at[idx_v.at[jj]], bufs[b], gsems[b]).start()

        def out_at(jj):
            return out_hbm.at[pl.ds((base + jj) * CH, CH)]

        for j in range(LOOK):
            gstart(j, j % NBUF)

        def step(to, carry):
            for b in range(NBUF):
                jj = to * NBUF + b
                ba = (b + LOOK) % NBUF
                ahead = jj + LOOK

                @pl.when(ahead < NCHUNK)
                def _():
                    @pl.when(ahead >= NBUF)
                    def _():
                        pltpu.make_async_copy(
                            bufs[ba], out_at(ahead - NBUF), ssems[ba]).wait()

                    pltpu.make_async_copy(
                        table_hbm.at[idx_v.at[ahead]], bufs[ba], gsems[ba]
                    ).start()

                pltpu.make_async_copy(
                    table_hbm.at[idx_v.at[jj]], bufs[b], gsems[b]).wait()
                pltpu.make_async_copy(bufs[b], out_at(jj), ssems[b]).start()
            return carry

        lax.fori_loop(0, NCHUNK // NBUF, step, 0)
        for j in range(NCHUNK - NBUF, NCHUNK):
            b = j % NBUF
            pltpu.make_async_copy(bufs[b], out_at(j), ssems[b]).wait()

    return gather_kernel(table, idx2)


def _tc_body(vf_ref, co_ref, g_ref, mk_ref, wq_ref, wk_ref, wv_ref, wo_ref,
             w1_ref, w2_ref, pw_ref, pv_ref, pb1_ref, seg_ref, out_ref):
    f32 = jnp.float32
    bf16 = jnp.bfloat16
    dims_tt = (((1,), (1,)), ((), ()))
    dims_nn = (((1,), (0,)), ((), ()))
    pv = pv_ref[...]
    seg = seg_ref[...]

    vf = vf_ref[...]
    q = lax.dot_general(vf.astype(bf16), wq_ref[...], dims_tt,
                        preferred_element_type=f32)
    q = (q + pv[0:1, :]) * (1.0 / math.sqrt(DH))

    coo = co_ref[...].reshape(KB, 3)
    pos = lax.dot_general(coo, pw_ref[...], dims_tt, preferred_element_type=f32)
    pos = jnp.maximum(pos + pv[8:9, :], 0.0).astype(bf16)

    g2 = g_ref[...].reshape(KB, CP)
    fb = pltpu.bitcast(g2, bf16).reshape(KB, C) + pos
    kp = lax.dot_general(fb, wk_ref[...], dims_tt, preferred_element_type=f32)
    vp = lax.dot_general(fb, wv_ref[...], dims_tt, preferred_element_type=f32)

    qk = jnp.broadcast_to(q[None, :, :], (K, BN, C)).reshape(KB, C)
    s = lax.dot_general(qk * kp, seg, dims_nn, preferred_element_type=f32)
    s3 = s.reshape(K, BN, H)
    s3 = jnp.where(mk_ref[...][:, :, None] > 0.5, -1e9, s3)
    m = jnp.max(s3, axis=0, keepdims=True)
    e = jnp.exp(s3 - m)
    attn = (e / jnp.sum(e, axis=0, keepdims=True)).reshape(KB, H)

    ab = lax.dot_general(attn, seg, (((1,), (1,)), ((), ())),
                         preferred_element_type=f32)
    mixed = jnp.sum((ab * vp).reshape(K, BN, C), axis=0)

    att = lax.dot_general((mixed + pv[1:2, :]).astype(bf16), wo_ref[...],
                          dims_tt, preferred_element_type=f32) + pv[2:3, :]

    def ln(x, g_row, b_row):
        mu = jnp.mean(x, axis=-1, keepdims=True)
        d = x - mu
        var = jnp.mean(d * d, axis=-1, keepdims=True)
        return d * lax.rsqrt(var + 1e-5) * g_row + b_row

    x = ln(vf + att, pv[4:5, :], pv[5:6, :])
    ff = jnp.maximum(
        lax.dot_general(x.astype(bf16), w1_ref[...], dims_tt,
                        preferred_element_type=f32)
        + pb1_ref[0:1, :], 0.0)
    f2 = lax.dot_general(ff.astype(bf16), w2_ref[...], dims_tt,
                         preferred_element_type=f32) + pv[3:4, :]
    out_ref[...] = ln(x + f2, pv[6:7, :], pv[7:8, :])


def kernel(voxel_features, key_coords, Wq, Wk, Wv, bq, bk, bv, Wo, bo,
           W1, b1, W2, b2, ln1_g, ln1_b, ln2_g, ln2_b, pos_w, pos_b,
           key_indices, key_mask):
    del bk

    vfb = voxel_features.astype(jnp.bfloat16)
    tb = lax.bitcast_convert_type(
        jnp.stack([vfb[:, :CP], vfb[:, CP:]], axis=-1), jnp.int32)

    idx_t = key_indices.T.astype(jnp.int32).reshape(NW, NCHUNK, CH)
    gathered = _sc_gather(tb, idx_t).reshape(K, N, CP)

    coords_t = key_coords.transpose(2, 0, 1)
    maskf = key_mask.T.astype(jnp.float32)
    zc = jnp.zeros((1, C), jnp.float32)
    pv = jnp.concatenate(
        [bq[None, :], bv[None, :], bo[None, :], b2[None, :],
         ln1_g[None, :], ln1_b[None, :], ln2_g[None, :], ln2_b[None, :],
         pos_b[None, :], zc, zc, zc, zc, zc, zc, zc], axis=0)
    pb1 = jnp.broadcast_to(b1[None, :], (8, FF))
    seg = (jnp.arange(C)[:, None] // DH ==
           jnp.arange(H)[None, :]).astype(jnp.float32)
    wkb = Wk.astype(jnp.bfloat16)
    wvb = Wv.astype(jnp.bfloat16)
    wqb = Wq.astype(jnp.bfloat16)
    wob = Wo.astype(jnp.bfloat16)
    w1b = W1.astype(jnp.bfloat16)
    w2b = W2.astype(jnp.bfloat16)

    grid = N // BN
    full = lambda shape: pl.BlockSpec(shape, lambda i: tuple(0 for _ in shape))
    out = pl.pallas_call(
        _tc_body,
        grid=(grid,),
        in_specs=[
            pl.BlockSpec((BN, C), lambda i: (i, 0)),
            pl.BlockSpec((K, BN, 3), lambda i: (0, i, 0)),
            pl.BlockSpec((K, BN, CP), lambda i: (0, i, 0)),
            pl.BlockSpec((K, BN), lambda i: (0, i)),
            full((C, C)), full((C, C)), full((C, C)), full((C, C)),
            full((FF, C)), full((C, FF)),
            full((C, 3)), full((16, C)), full((8, FF)), full((C, H)),
        ],
        out_specs=pl.BlockSpec((BN, C), lambda i: (i, 0)),
        out_shape=jax.ShapeDtypeStruct((N, C), jnp.float32),
        compiler_params=pltpu.CompilerParams(
            dimension_semantics=("arbitrary",)),
    )(voxel_features, coords_t, gathered, maskf, wqb, wkb, wvb, wob,
      w1b, w2b, pos_w, pv, pb1, seg)
    return out
